# baseline (speedup 1.0000x reference)
import jax
import jax.numpy as jnp
from jax.experimental import pallas as pl
from jax.experimental.pallas import tpu as pltpu

_N = 8192
_RES = 64
_C = 128
_P = 256
_NP = _N // _P
_W = 128


def _pass_big(x, j, k, desc):
    R, W = x.shape
    G = R // (2 * j)
    x3 = x.reshape(G, 2 * j, W)
    a = x3[:, :j, :]
    b = x3[:, j:, :]
    mn = jnp.minimum(a, b)
    mx = jnp.maximum(a, b)
    if desc:
        mn, mx = mx, mn
    dirs = [((g * 2 * j) & k) == 0 for g in range(G)]
    if all(dirs):
        first, second = mn, mx
    elif not any(dirs):
        first, second = mx, mn
    else:
        gi = jax.lax.broadcasted_iota(jnp.int32, (G, 1, 1), 0)
        dm = (gi & (k // (2 * j))) == 0
        first = jnp.where(dm, mn, mx)
        second = jnp.where(dm, mx, mn)
    out = jnp.concatenate([first[:, None], second[:, None]], axis=1)
    return out.reshape(R, W)


def _pass_small(x, j, k, desc):
    R, W = x.shape
    G = R // 8
    x3 = x.reshape(G, 8, W)
    s = jax.lax.broadcasted_iota(jnp.int32, (1, 8, 1), 1)
    up = (s & j) == 0
    p = jnp.where(up, pltpu.roll(x3, 8 - j, axis=1), pltpu.roll(x3, j, axis=1))
    mn = jnp.minimum(x3, p)
    mx = jnp.maximum(x3, p)
    if k < 8:
        take = up == ((s & k) == 0)
        if desc:
            take = jnp.logical_not(take)
        out = jnp.where(take, mn, mx)
    else:
        dirs = [((g * 8) & k) == 0 for g in range(G)]
        if all(d == dirs[0] for d in dirs):
            take = up if (dirs[0] != desc) else jnp.logical_not(up)
            out = jnp.where(take, mn, mx)
        else:
            gi = jax.lax.broadcasted_iota(jnp.int32, (G, 1, 1), 0)
            dm = (gi & (k // 8)) == 0
            if desc:
                dm = jnp.logical_not(dm)
            out = jnp.where(up == dm, mn, mx)
    return out.reshape(R, W)


def _net_pass(x, j, k, desc):
    if j >= 8:
        return _pass_big(x, j, k, desc)
    return _pass_small(x, j, k, desc)


def _local_sort(x, desc):
    R = x.shape[0]
    k = 2
    while k <= R:
        j = k // 2
        while j >= 1:
            x = _net_pass(x, j, k, desc)
            j //= 2
        k *= 2
    return x


def _merge_tail(x, desc):
    R = x.shape[0]
    j = R // 2
    while j >= 1:
        x = _net_pass(x, j, 2 * R, desc)
        j //= 2
    return x


def _sw_kernel(bT_ref, dT_ref, x_ref, y_ref, out_ref, buf):
    xb = x_ref[...]
    yb = y_ref[...]
    z = jnp.zeros_like(xb)
    w4 = jnp.concatenate(
        [
            jnp.concatenate([xb, z], axis=1),
            jnp.concatenate([z, xb], axis=1),
            jnp.concatenate([yb, z], axis=1),
            jnp.concatenate([z, yb], axis=1),
        ],
        axis=0,
    )
    bd = jnp.concatenate([bT_ref[0], dT_ref[0]], axis=1)
    buf[...] = jax.lax.dot_general(
        bd,
        w4,
        (((1,), (0,)), ((), ())),
        precision=jax.lax.Precision.HIGHEST,
        preferred_element_type=jnp.float32,
    )

    def _p1_body(p, desc):
        x = buf[pl.ds(p * _P, _P), :]
        lo = _local_sort(x[:_C, :], desc=False)
        hi = _local_sort(x[_C:, :], desc=True)
        mn = jnp.minimum(lo, hi)
        mx = jnp.maximum(lo, hi)
        if desc:
            mn, mx = mx, mn
        lo = _merge_tail(mn, desc)
        hi = _merge_tail(mx, desc)
        buf[pl.ds(p * _P, _P), :] = jnp.concatenate([lo, hi], axis=0)

    def p1_asc(u, carry):
        _p1_body(4 * u, False)
        _p1_body(4 * u + 2, False)
        return carry

    def p1_desc(u, carry):
        _p1_body(4 * u + 1, True)
        _p1_body(4 * u + 3, True)
        return carry

    jax.lax.fori_loop(0, _NP // 4, p1_asc, 0)
    jax.lax.fori_loop(0, _NP // 4, p1_desc, 0)

    k = 512
    while k <= _N:
        j = k // 2
        while j >= 2 * _P:
            jb = j // _P

            def _cross_one(u, j=j, jb=jb, k=k):
                g = u // jb
                t = u - g * jb
                base = g * (2 * j) + t * _P
                a = buf[pl.ds(base, _P), :]
                bq = buf[pl.ds(base + j, _P), :]
                mn = jnp.minimum(a, bq)
                mx = jnp.maximum(a, bq)
                asc = (base & k) == 0

                @pl.when(asc)
                def _():
                    buf[pl.ds(base, _P), :] = mn
                    buf[pl.ds(base + j, _P), :] = mx

                @pl.when(jnp.logical_not(asc))
                def _():
                    buf[pl.ds(base, _P), :] = mx
                    buf[pl.ds(base + j, _P), :] = mn

            def p2a_body(u, carry, cross=_cross_one):
                cross(2 * u)
                cross(2 * u + 1)
                return carry

            jax.lax.fori_loop(0, _N // (4 * _P), p2a_body, 0)
            j //= 2

        _Q = 2 * _P
        S = k // _Q
        last = k == _N

        def _tail_body(q, desc, last=last):
            base = q * _Q
            x = buf[pl.ds(base, _Q), :]
            a = x[:_P, :]
            b = x[_P:, :]
            mn = jnp.minimum(a, b)
            mx = jnp.maximum(a, b)
            if desc:
                mn, mx = mx, mn
            halves = []
            for h in (mn, mx):
                lo = h[:_C, :]
                hi = h[_C:, :]
                mn2 = jnp.minimum(lo, hi)
                mx2 = jnp.maximum(lo, hi)
                if desc:
                    mn2, mx2 = mx2, mn2
                halves.append(_merge_tail(mn2, desc))
                halves.append(_merge_tail(mx2, desc))
            res = jnp.concatenate(halves, axis=0)
            if last:
                out_ref[0, pl.ds(base, _Q), :] = res[:, :_RES]
                out_ref[1, pl.ds(base, _Q), :] = res[:, _RES:]
            else:
                buf[pl.ds(base, _Q), :] = res

        def _qmap(u, S=S):
            return (u // S) * 2 * S + (u - (u // S) * S)

        def p2b_asc(u, carry):
            _tail_body(_qmap(2 * u), False)
            _tail_body(_qmap(2 * u + 1), False)
            return carry

        def p2b_desc(u, carry):
            _tail_body(_qmap(2 * u) + S, True)
            _tail_body(_qmap(2 * u + 1) + S, True)
            return carry

        _NQ = _N // _Q
        if last:
            jax.lax.fori_loop(0, _NQ // 2, p2b_asc, 0)
        else:
            jax.lax.fori_loop(0, _NQ // 4, p2b_asc, 0)
            jax.lax.fori_loop(0, _NQ // 4, p2b_desc, 0)
        k *= 2


def kernel(b, d, x_basis, y_basis):
    bsz = b.shape[0]
    xr = x_basis.reshape(1, _RES)
    yr = y_basis.reshape(1, _RES)
    bT = b.reshape(bsz // 2, 2, _N).transpose(0, 2, 1)
    dT = d.reshape(bsz // 2, 2, _N).transpose(0, 2, 1)
    out = pl.pallas_call(
        _sw_kernel,
        grid=(bsz // 2,),
        in_specs=[
            pl.BlockSpec((1, _N, 2), lambda i: (i, 0, 0)),
            pl.BlockSpec((1, _N, 2), lambda i: (i, 0, 0)),
            pl.BlockSpec((1, _RES), lambda i: (0, 0)),
            pl.BlockSpec((1, _RES), lambda i: (0, 0)),
        ],
        out_specs=pl.BlockSpec((2, _N, _RES), lambda i: (i, 0, 0)),
        out_shape=jax.ShapeDtypeStruct((bsz, _N, _RES), jnp.float32),
        scratch_shapes=[pltpu.VMEM((_N, _W), jnp.float32)],
        compiler_params=pltpu.CompilerParams(
            dimension_semantics=("parallel",),
        ),
    )(bT, dT, xr, yr)
    return out


# speedup vs baseline: 1.0725x; 1.0725x over previous
import jax
import jax.numpy as jnp
from jax.experimental import pallas as pl
from jax.experimental.pallas import tpu as pltpu

_N = 8192
_RES = 64
_C = 128
_P = 256
_NP = _N // _P
_W = 128


def _pass_big(x, j, k, desc):
    R, W = x.shape
    G = R // (2 * j)
    x3 = x.reshape(G, 2 * j, W)
    a = x3[:, :j, :]
    b = x3[:, j:, :]
    mn = jnp.minimum(a, b)
    mx = jnp.maximum(a, b)
    if desc:
        mn, mx = mx, mn
    dirs = [((g * 2 * j) & k) == 0 for g in range(G)]
    if all(dirs):
        first, second = mn, mx
    elif not any(dirs):
        first, second = mx, mn
    else:
        gi = jax.lax.broadcasted_iota(jnp.int32, (G, 1, 1), 0)
        dm = (gi & (k // (2 * j))) == 0
        first = jnp.where(dm, mn, mx)
        second = jnp.where(dm, mx, mn)
    out = jnp.concatenate([first[:, None], second[:, None]], axis=1)
    return out.reshape(R, W)


def _pass_small(x, j, k, desc):
    R, W = x.shape
    G = R // 8
    x3 = x.reshape(G, 8, W)
    s = jax.lax.broadcasted_iota(jnp.int32, (1, 8, 1), 1)
    up = (s & j) == 0
    p = jnp.where(up, pltpu.roll(x3, 8 - j, axis=1), pltpu.roll(x3, j, axis=1))
    mn = jnp.minimum(x3, p)
    mx = jnp.maximum(x3, p)
    if k < 8:
        take = up == ((s & k) == 0)
        if desc:
            take = jnp.logical_not(take)
        out = jnp.where(take, mn, mx)
    else:
        dirs = [((g * 8) & k) == 0 for g in range(G)]
        if all(d == dirs[0] for d in dirs):
            take = up if (dirs[0] != desc) else jnp.logical_not(up)
            out = jnp.where(take, mn, mx)
        else:
            gi = jax.lax.broadcasted_iota(jnp.int32, (G, 1, 1), 0)
            dm = (gi & (k // 8)) == 0
            if desc:
                dm = jnp.logical_not(dm)
            out = jnp.where(up == dm, mn, mx)
    return out.reshape(R, W)


def _net_pass(x, j, k, desc):
    if j >= 8:
        return _pass_big(x, j, k, desc)
    return _pass_small(x, j, k, desc)


def _local_sort(x, desc):
    R = x.shape[0]
    k = 2
    while k <= R:
        j = k // 2
        while j >= 1:
            x = _net_pass(x, j, k, desc)
            j //= 2
        k *= 2
    return x


def _merge_tail(x, desc):
    R = x.shape[0]
    j = R // 2
    while j >= 1:
        x = _net_pass(x, j, 2 * R, desc)
        j //= 2
    return x


def _sw_kernel(bT_ref, dT_ref, x_ref, y_ref, out_ref, buf):
    xb = x_ref[...]
    yb = y_ref[...]
    z = jnp.zeros_like(xb)
    w4 = jnp.concatenate(
        [
            jnp.concatenate([xb, z], axis=1),
            jnp.concatenate([z, xb], axis=1),
            jnp.concatenate([yb, z], axis=1),
            jnp.concatenate([z, yb], axis=1),
        ],
        axis=0,
    )
    bd = jnp.concatenate([bT_ref[0], dT_ref[0]], axis=0)
    buf[...] = jax.lax.dot_general(
        bd,
        w4,
        (((0,), (0,)), ((), ())),
        precision=jax.lax.Precision.HIGHEST,
        preferred_element_type=jnp.float32,
    )

    def _p1_body(p, desc):
        x = buf[pl.ds(p * _P, _P), :]
        lo = _local_sort(x[:_C, :], desc=False)
        hi = _local_sort(x[_C:, :], desc=True)
        mn = jnp.minimum(lo, hi)
        mx = jnp.maximum(lo, hi)
        if desc:
            mn, mx = mx, mn
        lo = _merge_tail(mn, desc)
        hi = _merge_tail(mx, desc)
        buf[pl.ds(p * _P, _P), :] = jnp.concatenate([lo, hi], axis=0)

    def p1_asc(u, carry):
        _p1_body(4 * u, False)
        _p1_body(4 * u + 2, False)
        return carry

    def p1_desc(u, carry):
        _p1_body(4 * u + 1, True)
        _p1_body(4 * u + 3, True)
        return carry

    jax.lax.fori_loop(0, _NP // 4, p1_asc, 0)
    jax.lax.fori_loop(0, _NP // 4, p1_desc, 0)

    k = 512
    while k <= _N:
        j = k // 2
        while j >= 2 * _P:
            jb = j // _P

            def _cross_one(u, j=j, jb=jb, k=k):
                g = u // jb
                t = u - g * jb
                base = g * (2 * j) + t * _P
                a = buf[pl.ds(base, _P), :]
                bq = buf[pl.ds(base + j, _P), :]
                mn = jnp.minimum(a, bq)
                mx = jnp.maximum(a, bq)
                asc = (base & k) == 0

                @pl.when(asc)
                def _():
                    buf[pl.ds(base, _P), :] = mn
                    buf[pl.ds(base + j, _P), :] = mx

                @pl.when(jnp.logical_not(asc))
                def _():
                    buf[pl.ds(base, _P), :] = mx
                    buf[pl.ds(base + j, _P), :] = mn

            def p2a_body(u, carry, cross=_cross_one):
                cross(2 * u)
                cross(2 * u + 1)
                return carry

            jax.lax.fori_loop(0, _N // (4 * _P), p2a_body, 0)
            j //= 2

        _Q = 2 * _P
        S = k // _Q
        last = k == _N

        def _tail_body(q, desc, last=last):
            base = q * _Q
            x = buf[pl.ds(base, _Q), :]
            a = x[:_P, :]
            b = x[_P:, :]
            mn = jnp.minimum(a, b)
            mx = jnp.maximum(a, b)
            if desc:
                mn, mx = mx, mn
            halves = []
            for h in (mn, mx):
                lo = h[:_C, :]
                hi = h[_C:, :]
                mn2 = jnp.minimum(lo, hi)
                mx2 = jnp.maximum(lo, hi)
                if desc:
                    mn2, mx2 = mx2, mn2
                halves.append(_merge_tail(mn2, desc))
                halves.append(_merge_tail(mx2, desc))
            res = jnp.concatenate(halves, axis=0)
            if last:
                out_ref[0, pl.ds(base, _Q), :] = res[:, :_RES]
                out_ref[1, pl.ds(base, _Q), :] = res[:, _RES:]
            else:
                buf[pl.ds(base, _Q), :] = res

        def _qmap(u, S=S):
            return (u // S) * 2 * S + (u - (u // S) * S)

        def p2b_asc(u, carry):
            _tail_body(_qmap(2 * u), False)
            _tail_body(_qmap(2 * u + 1), False)
            return carry

        def p2b_desc(u, carry):
            _tail_body(_qmap(2 * u) + S, True)
            _tail_body(_qmap(2 * u + 1) + S, True)
            return carry

        _NQ = _N // _Q
        if last:
            jax.lax.fori_loop(0, _NQ // 2, p2b_asc, 0)
        else:
            jax.lax.fori_loop(0, _NQ // 4, p2b_asc, 0)
            jax.lax.fori_loop(0, _NQ // 4, p2b_desc, 0)
        k *= 2


def kernel(b, d, x_basis, y_basis):
    bsz = b.shape[0]
    xr = x_basis.reshape(1, _RES)
    yr = y_basis.reshape(1, _RES)
    bT = b.reshape(bsz // 2, 2, _N)
    dT = d.reshape(bsz // 2, 2, _N)
    out = pl.pallas_call(
        _sw_kernel,
        grid=(bsz // 2,),
        in_specs=[
            pl.BlockSpec((1, 2, _N), lambda i: (i, 0, 0)),
            pl.BlockSpec((1, 2, _N), lambda i: (i, 0, 0)),
            pl.BlockSpec((1, _RES), lambda i: (0, 0)),
            pl.BlockSpec((1, _RES), lambda i: (0, 0)),
        ],
        out_specs=pl.BlockSpec((2, _N, _RES), lambda i: (i, 0, 0)),
        out_shape=jax.ShapeDtypeStruct((bsz, _N, _RES), jnp.float32),
        scratch_shapes=[pltpu.VMEM((_N, _W), jnp.float32)],
        compiler_params=pltpu.CompilerParams(
            dimension_semantics=("parallel",),
        ),
    )(bT, dT, xr, yr)
    return out


# speedup vs baseline: 1.1195x; 1.0438x over previous
import jax
import jax.numpy as jnp
from jax.experimental import pallas as pl
from jax.experimental.pallas import tpu as pltpu

_N = 8192
_RES = 64
_C = 128
_P = 256
_NP = _N // _P
_W = 128


def _pass_big(x, j, k, desc):
    R, W = x.shape
    G = R // (2 * j)
    x3 = x.reshape(G, 2 * j, W)
    a = x3[:, :j, :]
    b = x3[:, j:, :]
    mn = jnp.minimum(a, b)
    mx = jnp.maximum(a, b)
    if desc:
        mn, mx = mx, mn
    dirs = [((g * 2 * j) & k) == 0 for g in range(G)]
    if all(dirs):
        first, second = mn, mx
    elif not any(dirs):
        first, second = mx, mn
    else:
        gi = jax.lax.broadcasted_iota(jnp.int32, (G, 1, 1), 0)
        dm = (gi & (k // (2 * j))) == 0
        first = jnp.where(dm, mn, mx)
        second = jnp.where(dm, mx, mn)
    out = jnp.concatenate([first[:, None], second[:, None]], axis=1)
    return out.reshape(R, W)


def _pass_small(x, j, k, desc):
    R, W = x.shape
    G = R // 8
    x3 = x.reshape(G, 8, W)
    s = jax.lax.broadcasted_iota(jnp.int32, (1, 8, 1), 1)
    up = (s & j) == 0
    p = jnp.where(up, pltpu.roll(x3, 8 - j, axis=1), pltpu.roll(x3, j, axis=1))
    mn = jnp.minimum(x3, p)
    mx = jnp.maximum(x3, p)
    if k < 8:
        take = up == ((s & k) == 0)
        if desc:
            take = jnp.logical_not(take)
        out = jnp.where(take, mn, mx)
    else:
        dirs = [((g * 8) & k) == 0 for g in range(G)]
        if all(d == dirs[0] for d in dirs):
            take = up if (dirs[0] != desc) else jnp.logical_not(up)
            out = jnp.where(take, mn, mx)
        else:
            gi = jax.lax.broadcasted_iota(jnp.int32, (G, 1, 1), 0)
            dm = (gi & (k // 8)) == 0
            if desc:
                dm = jnp.logical_not(dm)
            out = jnp.where(up == dm, mn, mx)
    return out.reshape(R, W)


def _net_pass(x, j, k, desc):
    if j >= 8:
        return _pass_big(x, j, k, desc)
    return _pass_small(x, j, k, desc)


def _local_sort(x, desc):
    R = x.shape[0]
    k = 2
    while k <= R:
        j = k // 2
        while j >= 1:
            x = _net_pass(x, j, k, desc)
            j //= 2
        k *= 2
    return x


def _merge_tail(x, desc):
    R = x.shape[0]
    j = R // 2
    while j >= 1:
        x = _net_pass(x, j, 2 * R, desc)
        j //= 2
    return x


def _sw_kernel(bT_ref, dT_ref, x_ref, y_ref, out_ref, buf):
    xb = x_ref[...]
    yb = y_ref[...]
    z = jnp.zeros_like(xb)
    w4 = jnp.concatenate(
        [
            jnp.concatenate([xb, z], axis=1),
            jnp.concatenate([z, xb], axis=1),
            jnp.concatenate([yb, z], axis=1),
            jnp.concatenate([z, yb], axis=1),
        ],
        axis=0,
    )
    bd = jnp.concatenate([bT_ref[0], dT_ref[0]], axis=0)
    buf[...] = jax.lax.dot_general(
        bd,
        w4,
        (((0,), (0,)), ((), ())),
        precision=jax.lax.Precision.HIGHEST,
        preferred_element_type=jnp.float32,
    )

    def _p1_body(p, desc):
        x = buf[pl.ds(p * _P, _P), :]
        lo = _local_sort(x[:_C, :], desc=False)
        hi = _local_sort(x[_C:, :], desc=True)
        mn = jnp.minimum(lo, hi)
        mx = jnp.maximum(lo, hi)
        if desc:
            mn, mx = mx, mn
        lo = _merge_tail(mn, desc)
        hi = _merge_tail(mx, desc)
        buf[pl.ds(p * _P, _P), :] = jnp.concatenate([lo, hi], axis=0)

    def p1_asc(u, carry):
        _p1_body(4 * u, False)
        _p1_body(4 * u + 2, False)
        return carry

    def p1_desc(u, carry):
        _p1_body(4 * u + 1, True)
        _p1_body(4 * u + 3, True)
        return carry

    jax.lax.fori_loop(0, _NP // 4, p1_asc, 0)
    jax.lax.fori_loop(0, _NP // 4, p1_desc, 0)

    k = 512
    while k <= _N:
        j = k // 2
        while j >= 4 * _P:
            j2 = j // 2
            m = j2 // _P

            def p2f_body(u, carry, j=j, j2=j2, m=m, k=k):
                g = u // m
                t = u - g * m
                base = g * (2 * j) + t * _P
                A = buf[pl.ds(base, _P), :]
                B = buf[pl.ds(base + j2, _P), :]
                C = buf[pl.ds(base + j, _P), :]
                D = buf[pl.ds(base + j + j2, _P), :]
                mnAC = jnp.minimum(A, C)
                mxAC = jnp.maximum(A, C)
                mnBD = jnp.minimum(B, D)
                mxBD = jnp.maximum(B, D)
                a2 = jnp.minimum(mnAC, mnBD)
                b2 = jnp.maximum(mnAC, mnBD)
                c2 = jnp.minimum(mxAC, mxBD)
                d2 = jnp.maximum(mxAC, mxBD)
                asc = (base & k) == 0

                @pl.when(asc)
                def _():
                    buf[pl.ds(base, _P), :] = a2
                    buf[pl.ds(base + j2, _P), :] = b2
                    buf[pl.ds(base + j, _P), :] = c2
                    buf[pl.ds(base + j + j2, _P), :] = d2

                @pl.when(jnp.logical_not(asc))
                def _():
                    buf[pl.ds(base, _P), :] = d2
                    buf[pl.ds(base + j2, _P), :] = c2
                    buf[pl.ds(base + j, _P), :] = b2
                    buf[pl.ds(base + j + j2, _P), :] = a2

                return carry

            jax.lax.fori_loop(0, _N // (4 * _P), p2f_body, 0)
            j //= 4
        while j >= 2 * _P:
            jb = j // _P

            def _cross_one(u, j=j, jb=jb, k=k):
                g = u // jb
                t = u - g * jb
                base = g * (2 * j) + t * _P
                a = buf[pl.ds(base, _P), :]
                bq = buf[pl.ds(base + j, _P), :]
                mn = jnp.minimum(a, bq)
                mx = jnp.maximum(a, bq)
                asc = (base & k) == 0

                @pl.when(asc)
                def _():
                    buf[pl.ds(base, _P), :] = mn
                    buf[pl.ds(base + j, _P), :] = mx

                @pl.when(jnp.logical_not(asc))
                def _():
                    buf[pl.ds(base, _P), :] = mx
                    buf[pl.ds(base + j, _P), :] = mn

            def p2a_body(u, carry, cross=_cross_one):
                cross(2 * u)
                cross(2 * u + 1)
                return carry

            jax.lax.fori_loop(0, _N // (4 * _P), p2a_body, 0)
            j //= 2

        _Q = 2 * _P
        S = k // _Q
        last = k == _N

        def _tail_body(q, desc, last=last):
            base = q * _Q
            x = buf[pl.ds(base, _Q), :]
            a = x[:_P, :]
            b = x[_P:, :]
            mn = jnp.minimum(a, b)
            mx = jnp.maximum(a, b)
            if desc:
                mn, mx = mx, mn
            halves = []
            for h in (mn, mx):
                lo = h[:_C, :]
                hi = h[_C:, :]
                mn2 = jnp.minimum(lo, hi)
                mx2 = jnp.maximum(lo, hi)
                if desc:
                    mn2, mx2 = mx2, mn2
                halves.append(_merge_tail(mn2, desc))
                halves.append(_merge_tail(mx2, desc))
            res = jnp.concatenate(halves, axis=0)
            if last:
                out_ref[0, pl.ds(base, _Q), :] = res[:, :_RES]
                out_ref[1, pl.ds(base, _Q), :] = res[:, _RES:]
            else:
                buf[pl.ds(base, _Q), :] = res

        def _qmap(u, S=S):
            return (u // S) * 2 * S + (u - (u // S) * S)

        def p2b_asc(u, carry):
            _tail_body(_qmap(2 * u), False)
            _tail_body(_qmap(2 * u + 1), False)
            return carry

        def p2b_desc(u, carry):
            _tail_body(_qmap(2 * u) + S, True)
            _tail_body(_qmap(2 * u + 1) + S, True)
            return carry

        _NQ = _N // _Q
        if last:
            jax.lax.fori_loop(0, _NQ // 2, p2b_asc, 0)
        else:
            jax.lax.fori_loop(0, _NQ // 4, p2b_asc, 0)
            jax.lax.fori_loop(0, _NQ // 4, p2b_desc, 0)
        k *= 2


def kernel(b, d, x_basis, y_basis):
    bsz = b.shape[0]
    xr = x_basis.reshape(1, _RES)
    yr = y_basis.reshape(1, _RES)
    bT = b.reshape(bsz // 2, 2, _N)
    dT = d.reshape(bsz // 2, 2, _N)
    out = pl.pallas_call(
        _sw_kernel,
        grid=(bsz // 2,),
        in_specs=[
            pl.BlockSpec((1, 2, _N), lambda i: (i, 0, 0)),
            pl.BlockSpec((1, 2, _N), lambda i: (i, 0, 0)),
            pl.BlockSpec((1, _RES), lambda i: (0, 0)),
            pl.BlockSpec((1, _RES), lambda i: (0, 0)),
        ],
        out_specs=pl.BlockSpec((2, _N, _RES), lambda i: (i, 0, 0)),
        out_shape=jax.ShapeDtypeStruct((bsz, _N, _RES), jnp.float32),
        scratch_shapes=[pltpu.VMEM((_N, _W), jnp.float32)],
        compiler_params=pltpu.CompilerParams(
            dimension_semantics=("parallel",),
        ),
    )(bT, dT, xr, yr)
    return out


# speedup vs baseline: 1.1632x; 1.0391x over previous
import jax
import jax.numpy as jnp
from jax.experimental import pallas as pl
from jax.experimental.pallas import tpu as pltpu

_N = 8192
_RES = 64
_C = 128
_P = 256
_NP = _N // _P
_W = 128


def _pass_big(x, j, k, desc):
    R, W = x.shape
    G = R // (2 * j)
    x3 = x.reshape(G, 2 * j, W)
    a = x3[:, :j, :]
    b = x3[:, j:, :]
    mn = jnp.minimum(a, b)
    mx = jnp.maximum(a, b)
    if desc:
        mn, mx = mx, mn
    dirs = [((g * 2 * j) & k) == 0 for g in range(G)]
    if all(dirs):
        first, second = mn, mx
    elif not any(dirs):
        first, second = mx, mn
    else:
        gi = jax.lax.broadcasted_iota(jnp.int32, (G, 1, 1), 0)
        dm = (gi & (k // (2 * j))) == 0
        first = jnp.where(dm, mn, mx)
        second = jnp.where(dm, mx, mn)
    out = jnp.concatenate([first[:, None], second[:, None]], axis=1)
    return out.reshape(R, W)


def _pass_small(x, j, k, desc):
    R, W = x.shape
    G = R // 8
    x3 = x.reshape(G, 8, W)
    s = jax.lax.broadcasted_iota(jnp.int32, (1, 8, 1), 1)
    up = (s & j) == 0
    uniform = k >= 8 and all(
        (((g * 8) & k) == 0) == (((0 * 8) & k) == 0) for g in range(G)
    )
    if j == 4:
        p = pltpu.roll(x3, 4, axis=1)
        mn = jnp.minimum(x3, p)
        mx = jnp.maximum(x3, p)
        if uniform:
            asc0 = (((0) & k) == 0) != desc
            take = up if asc0 else jnp.logical_not(up)
            out = jnp.where(take, mn, mx)
        else:
            gi = jax.lax.broadcasted_iota(jnp.int32, (G, 1, 1), 0)
            dm = (gi & (k // 8)) == 0
            if desc:
                dm = jnp.logical_not(dm)
            out = jnp.where(up == dm, mn, mx)
        return out.reshape(R, W)
    if uniform:
        r = pltpu.roll(x3, 8 - j, axis=1)
        mn = jnp.minimum(x3, r)
        mx = jnp.maximum(x3, r)
        if desc:
            mn, mx = mx, mn
        out = jnp.where(up, mn, pltpu.roll(mx, j, axis=1))
        return out.reshape(R, W)
    p = jnp.where(up, pltpu.roll(x3, 8 - j, axis=1), pltpu.roll(x3, j, axis=1))
    mn = jnp.minimum(x3, p)
    mx = jnp.maximum(x3, p)
    if k < 8:
        take = up == ((s & k) == 0)
        if desc:
            take = jnp.logical_not(take)
        out = jnp.where(take, mn, mx)
    else:
        gi = jax.lax.broadcasted_iota(jnp.int32, (G, 1, 1), 0)
        dm = (gi & (k // 8)) == 0
        if desc:
            dm = jnp.logical_not(dm)
        out = jnp.where(up == dm, mn, mx)
    return out.reshape(R, W)


def _net_pass(x, j, k, desc):
    if j >= 8:
        return _pass_big(x, j, k, desc)
    return _pass_small(x, j, k, desc)


def _local_sort(x, desc):
    R = x.shape[0]
    k = 2
    while k <= R:
        j = k // 2
        while j >= 1:
            x = _net_pass(x, j, k, desc)
            j //= 2
        k *= 2
    return x


def _merge_tail(x, desc):
    R = x.shape[0]
    j = R // 2
    while j >= 1:
        x = _net_pass(x, j, 2 * R, desc)
        j //= 2
    return x


def _sw_kernel(bT_ref, dT_ref, x_ref, y_ref, out_ref, buf):
    xb = x_ref[...]
    yb = y_ref[...]
    z = jnp.zeros_like(xb)
    w4 = jnp.concatenate(
        [
            jnp.concatenate([xb, z], axis=1),
            jnp.concatenate([z, xb], axis=1),
            jnp.concatenate([yb, z], axis=1),
            jnp.concatenate([z, yb], axis=1),
        ],
        axis=0,
    )
    bd = jnp.concatenate([bT_ref[0], dT_ref[0]], axis=0)
    buf[...] = jax.lax.dot_general(
        bd,
        w4,
        (((0,), (0,)), ((), ())),
        precision=jax.lax.Precision.HIGHEST,
        preferred_element_type=jnp.float32,
    )

    def _p1_body(p, desc):
        x = buf[pl.ds(p * _P, _P), :]
        lo = _local_sort(x[:_C, :], desc=False)
        hi = _local_sort(x[_C:, :], desc=True)
        mn = jnp.minimum(lo, hi)
        mx = jnp.maximum(lo, hi)
        if desc:
            mn, mx = mx, mn
        lo = _merge_tail(mn, desc)
        hi = _merge_tail(mx, desc)
        buf[pl.ds(p * _P, _P), :] = jnp.concatenate([lo, hi], axis=0)

    def p1_asc(u, carry):
        _p1_body(4 * u, False)
        _p1_body(4 * u + 2, False)
        return carry

    def p1_desc(u, carry):
        _p1_body(4 * u + 1, True)
        _p1_body(4 * u + 3, True)
        return carry

    jax.lax.fori_loop(0, _NP // 4, p1_asc, 0)
    jax.lax.fori_loop(0, _NP // 4, p1_desc, 0)

    k = 512
    while k <= _N:
        j = k // 2
        while j >= 4 * _P:
            j2 = j // 2
            m = j2 // _P

            def p2f_body(u, carry, j=j, j2=j2, m=m, k=k):
                g = u // m
                t = u - g * m
                base = g * (2 * j) + t * _P
                A = buf[pl.ds(base, _P), :]
                B = buf[pl.ds(base + j2, _P), :]
                C = buf[pl.ds(base + j, _P), :]
                D = buf[pl.ds(base + j + j2, _P), :]
                mnAC = jnp.minimum(A, C)
                mxAC = jnp.maximum(A, C)
                mnBD = jnp.minimum(B, D)
                mxBD = jnp.maximum(B, D)
                a2 = jnp.minimum(mnAC, mnBD)
                b2 = jnp.maximum(mnAC, mnBD)
                c2 = jnp.minimum(mxAC, mxBD)
                d2 = jnp.maximum(mxAC, mxBD)
                asc = (base & k) == 0

                @pl.when(asc)
                def _():
                    buf[pl.ds(base, _P), :] = a2
                    buf[pl.ds(base + j2, _P), :] = b2
                    buf[pl.ds(base + j, _P), :] = c2
                    buf[pl.ds(base + j + j2, _P), :] = d2

                @pl.when(jnp.logical_not(asc))
                def _():
                    buf[pl.ds(base, _P), :] = d2
                    buf[pl.ds(base + j2, _P), :] = c2
                    buf[pl.ds(base + j, _P), :] = b2
                    buf[pl.ds(base + j + j2, _P), :] = a2

                return carry

            jax.lax.fori_loop(0, _N // (4 * _P), p2f_body, 0)
            j //= 4
        while j >= 2 * _P:
            jb = j // _P

            def _cross_one(u, j=j, jb=jb, k=k):
                g = u // jb
                t = u - g * jb
                base = g * (2 * j) + t * _P
                a = buf[pl.ds(base, _P), :]
                bq = buf[pl.ds(base + j, _P), :]
                mn = jnp.minimum(a, bq)
                mx = jnp.maximum(a, bq)
                asc = (base & k) == 0

                @pl.when(asc)
                def _():
                    buf[pl.ds(base, _P), :] = mn
                    buf[pl.ds(base + j, _P), :] = mx

                @pl.when(jnp.logical_not(asc))
                def _():
                    buf[pl.ds(base, _P), :] = mx
                    buf[pl.ds(base + j, _P), :] = mn

            def p2a_body(u, carry, cross=_cross_one):
                cross(2 * u)
                cross(2 * u + 1)
                return carry

            jax.lax.fori_loop(0, _N // (4 * _P), p2a_body, 0)
            j //= 2

        _Q = 2 * _P
        S = k // _Q
        last = k == _N

        def _tail_body(q, desc, last=last):
            base = q * _Q
            x = buf[pl.ds(base, _Q), :]
            a = x[:_P, :]
            b = x[_P:, :]
            mn = jnp.minimum(a, b)
            mx = jnp.maximum(a, b)
            if desc:
                mn, mx = mx, mn
            halves = []
            for h in (mn, mx):
                lo = h[:_C, :]
                hi = h[_C:, :]
                mn2 = jnp.minimum(lo, hi)
                mx2 = jnp.maximum(lo, hi)
                if desc:
                    mn2, mx2 = mx2, mn2
                halves.append(_merge_tail(mn2, desc))
                halves.append(_merge_tail(mx2, desc))
            res = jnp.concatenate(halves, axis=0)
            if last:
                out_ref[0, pl.ds(base, _Q), :] = res[:, :_RES]
                out_ref[1, pl.ds(base, _Q), :] = res[:, _RES:]
            else:
                buf[pl.ds(base, _Q), :] = res

        def _qmap(u, S=S):
            return (u // S) * 2 * S + (u - (u // S) * S)

        def p2b_asc(u, carry):
            _tail_body(_qmap(2 * u), False)
            _tail_body(_qmap(2 * u + 1), False)
            return carry

        def p2b_desc(u, carry):
            _tail_body(_qmap(2 * u) + S, True)
            _tail_body(_qmap(2 * u + 1) + S, True)
            return carry

        _NQ = _N // _Q
        if last:
            jax.lax.fori_loop(0, _NQ // 2, p2b_asc, 0)
        else:
            jax.lax.fori_loop(0, _NQ // 4, p2b_asc, 0)
            jax.lax.fori_loop(0, _NQ // 4, p2b_desc, 0)
        k *= 2


def kernel(b, d, x_basis, y_basis):
    bsz = b.shape[0]
    xr = x_basis.reshape(1, _RES)
    yr = y_basis.reshape(1, _RES)
    bT = b.reshape(bsz // 2, 2, _N)
    dT = d.reshape(bsz // 2, 2, _N)
    out = pl.pallas_call(
        _sw_kernel,
        grid=(bsz // 2,),
        in_specs=[
            pl.BlockSpec((1, 2, _N), lambda i: (i, 0, 0)),
            pl.BlockSpec((1, 2, _N), lambda i: (i, 0, 0)),
            pl.BlockSpec((1, _RES), lambda i: (0, 0)),
            pl.BlockSpec((1, _RES), lambda i: (0, 0)),
        ],
        out_specs=pl.BlockSpec((2, _N, _RES), lambda i: (i, 0, 0)),
        out_shape=jax.ShapeDtypeStruct((bsz, _N, _RES), jnp.float32),
        scratch_shapes=[pltpu.VMEM((_N, _W), jnp.float32)],
        compiler_params=pltpu.CompilerParams(
            dimension_semantics=("parallel",),
        ),
    )(bT, dT, xr, yr)
    return out


# speedup vs baseline: 1.1653x; 1.0017x over previous
import jax
import jax.numpy as jnp
from jax.experimental import pallas as pl
from jax.experimental.pallas import tpu as pltpu

_N = 8192
_RES = 64
_C = 128
_P = 256
_NP = _N // _P
_W = 128


def _pass_big(x, j, k, desc):
    R, W = x.shape
    G = R // (2 * j)
    x3 = x.reshape(G, 2 * j, W)
    a = x3[:, :j, :]
    b = x3[:, j:, :]
    mn = jnp.minimum(a, b)
    mx = jnp.maximum(a, b)
    if desc:
        mn, mx = mx, mn
    dirs = [((g * 2 * j) & k) == 0 for g in range(G)]
    if all(dirs):
        first, second = mn, mx
    elif not any(dirs):
        first, second = mx, mn
    else:
        gi = jax.lax.broadcasted_iota(jnp.int32, (G, 1, 1), 0)
        dm = (gi & (k // (2 * j))) == 0
        first = jnp.where(dm, mn, mx)
        second = jnp.where(dm, mx, mn)
    out = jnp.concatenate([first[:, None], second[:, None]], axis=1)
    return out.reshape(R, W)


def _pass_small(x, j, k, desc):
    R, W = x.shape
    G = R // 8
    x3 = x.reshape(G, 8, W)
    s = jax.lax.broadcasted_iota(jnp.int32, (1, 8, 1), 1)
    up = (s & j) == 0
    uniform = k >= 8 and all(
        (((g * 8) & k) == 0) == (((0 * 8) & k) == 0) for g in range(G)
    )
    if j == 4:
        p = pltpu.roll(x3, 4, axis=1)
        mn = jnp.minimum(x3, p)
        mx = jnp.maximum(x3, p)
        if uniform:
            asc0 = (((0) & k) == 0) != desc
            take = up if asc0 else jnp.logical_not(up)
            out = jnp.where(take, mn, mx)
        else:
            gi = jax.lax.broadcasted_iota(jnp.int32, (G, 1, 1), 0)
            dm = (gi & (k // 8)) == 0
            if desc:
                dm = jnp.logical_not(dm)
            out = jnp.where(up == dm, mn, mx)
        return out.reshape(R, W)
    if uniform:
        r = pltpu.roll(x3, 8 - j, axis=1)
        mn = jnp.minimum(x3, r)
        mx = jnp.maximum(x3, r)
        if desc:
            mn, mx = mx, mn
        out = jnp.where(up, mn, pltpu.roll(mx, j, axis=1))
        return out.reshape(R, W)
    p = jnp.where(up, pltpu.roll(x3, 8 - j, axis=1), pltpu.roll(x3, j, axis=1))
    mn = jnp.minimum(x3, p)
    mx = jnp.maximum(x3, p)
    if k < 8:
        take = up == ((s & k) == 0)
        if desc:
            take = jnp.logical_not(take)
        out = jnp.where(take, mn, mx)
    else:
        gi = jax.lax.broadcasted_iota(jnp.int32, (G, 1, 1), 0)
        dm = (gi & (k // 8)) == 0
        if desc:
            dm = jnp.logical_not(dm)
        out = jnp.where(up == dm, mn, mx)
    return out.reshape(R, W)


def _net_pass(x, j, k, desc):
    if j >= 8:
        return _pass_big(x, j, k, desc)
    return _pass_small(x, j, k, desc)


def _local_sort(x, desc):
    R = x.shape[0]
    k = 2
    while k <= R:
        j = k // 2
        while j >= 1:
            x = _net_pass(x, j, k, desc)
            j //= 2
        k *= 2
    return x


def _merge_tail(x, desc):
    R = x.shape[0]
    j = R // 2
    while j >= 1:
        x = _net_pass(x, j, 2 * R, desc)
        j //= 2
    return x


def _sw_kernel(bT_ref, dT_ref, x_ref, y_ref, out_ref, buf):
    xb = x_ref[...]
    yb = y_ref[...]
    z = jnp.zeros_like(xb)
    w4 = jnp.concatenate(
        [
            jnp.concatenate([xb, z], axis=1),
            jnp.concatenate([z, xb], axis=1),
            jnp.concatenate([yb, z], axis=1),
            jnp.concatenate([z, yb], axis=1),
        ],
        axis=0,
    )
    bd = jnp.concatenate([bT_ref[0], dT_ref[0]], axis=0)
    buf[...] = jax.lax.dot_general(
        bd,
        w4,
        (((0,), (0,)), ((), ())),
        precision=jax.lax.Precision.HIGHEST,
        preferred_element_type=jnp.float32,
    )

    def _p1_body(p, desc):
        x = buf[pl.ds(p * _P, _P), :]
        lo = _local_sort(x[:_C, :], desc=False)
        hi = _local_sort(x[_C:, :], desc=True)
        mn = jnp.minimum(lo, hi)
        mx = jnp.maximum(lo, hi)
        if desc:
            mn, mx = mx, mn
        lo = _merge_tail(mn, desc)
        hi = _merge_tail(mx, desc)
        buf[pl.ds(p * _P, _P), :] = jnp.concatenate([lo, hi], axis=0)

    def p1_asc(u, carry):
        _p1_body(4 * u, False)
        _p1_body(4 * u + 2, False)
        return carry

    def p1_desc(u, carry):
        _p1_body(4 * u + 1, True)
        _p1_body(4 * u + 3, True)
        return carry

    jax.lax.fori_loop(0, _NP // 4, p1_asc, 0)
    jax.lax.fori_loop(0, _NP // 4, p1_desc, 0)

    k = 512
    while k <= _N:
        j = k // 2
        while j >= 4 * _P:
            j2 = j // 2
            m = j2 // _P

            def _p2f_one(u, j, j2, m, k):
                g = u // m
                t = u - g * m
                base = g * (2 * j) + t * _P
                A = buf[pl.ds(base, _P), :]
                B = buf[pl.ds(base + j2, _P), :]
                C = buf[pl.ds(base + j, _P), :]
                D = buf[pl.ds(base + j + j2, _P), :]
                mnAC = jnp.minimum(A, C)
                mxAC = jnp.maximum(A, C)
                mnBD = jnp.minimum(B, D)
                mxBD = jnp.maximum(B, D)
                a2 = jnp.minimum(mnAC, mnBD)
                b2 = jnp.maximum(mnAC, mnBD)
                c2 = jnp.minimum(mxAC, mxBD)
                d2 = jnp.maximum(mxAC, mxBD)
                asc = (base & k) == 0

                @pl.when(asc)
                def _():
                    buf[pl.ds(base, _P), :] = a2
                    buf[pl.ds(base + j2, _P), :] = b2
                    buf[pl.ds(base + j, _P), :] = c2
                    buf[pl.ds(base + j + j2, _P), :] = d2

                @pl.when(jnp.logical_not(asc))
                def _():
                    buf[pl.ds(base, _P), :] = d2
                    buf[pl.ds(base + j2, _P), :] = c2
                    buf[pl.ds(base + j, _P), :] = b2
                    buf[pl.ds(base + j + j2, _P), :] = a2

            def p2f_body(u, carry, j=j, j2=j2, m=m, k=k):
                _p2f_one(2 * u, j, j2, m, k)
                _p2f_one(2 * u + 1, j, j2, m, k)
                return carry

            jax.lax.fori_loop(0, _N // (8 * _P), p2f_body, 0)
            j //= 4
        while j >= 2 * _P:
            jb = j // _P

            def _cross_one(u, j=j, jb=jb, k=k):
                g = u // jb
                t = u - g * jb
                base = g * (2 * j) + t * _P
                a = buf[pl.ds(base, _P), :]
                bq = buf[pl.ds(base + j, _P), :]
                mn = jnp.minimum(a, bq)
                mx = jnp.maximum(a, bq)
                asc = (base & k) == 0

                @pl.when(asc)
                def _():
                    buf[pl.ds(base, _P), :] = mn
                    buf[pl.ds(base + j, _P), :] = mx

                @pl.when(jnp.logical_not(asc))
                def _():
                    buf[pl.ds(base, _P), :] = mx
                    buf[pl.ds(base + j, _P), :] = mn

            def p2a_body(u, carry, cross=_cross_one):
                cross(2 * u)
                cross(2 * u + 1)
                return carry

            jax.lax.fori_loop(0, _N // (4 * _P), p2a_body, 0)
            j //= 2

        _Q = 2 * _P
        S = k // _Q
        last = k == _N

        def _tail_body(q, desc, last=last):
            base = q * _Q
            x = buf[pl.ds(base, _Q), :]
            a = x[:_P, :]
            b = x[_P:, :]
            mn = jnp.minimum(a, b)
            mx = jnp.maximum(a, b)
            if desc:
                mn, mx = mx, mn
            halves = []
            for h in (mn, mx):
                lo = h[:_C, :]
                hi = h[_C:, :]
                mn2 = jnp.minimum(lo, hi)
                mx2 = jnp.maximum(lo, hi)
                if desc:
                    mn2, mx2 = mx2, mn2
                halves.append(_merge_tail(mn2, desc))
                halves.append(_merge_tail(mx2, desc))
            res = jnp.concatenate(halves, axis=0)
            if last:
                out_ref[0, pl.ds(base, _Q), :] = res[:, :_RES]
                out_ref[1, pl.ds(base, _Q), :] = res[:, _RES:]
            else:
                buf[pl.ds(base, _Q), :] = res

        def _qmap(u, S=S):
            return (u // S) * 2 * S + (u - (u // S) * S)

        def p2b_asc(u, carry):
            _tail_body(_qmap(2 * u), False)
            _tail_body(_qmap(2 * u + 1), False)
            return carry

        def p2b_desc(u, carry):
            _tail_body(_qmap(2 * u) + S, True)
            _tail_body(_qmap(2 * u + 1) + S, True)
            return carry

        _NQ = _N // _Q
        if last:
            jax.lax.fori_loop(0, _NQ // 2, p2b_asc, 0)
        else:
            jax.lax.fori_loop(0, _NQ // 4, p2b_asc, 0)
            jax.lax.fori_loop(0, _NQ // 4, p2b_desc, 0)
        k *= 2


def kernel(b, d, x_basis, y_basis):
    bsz = b.shape[0]
    xr = x_basis.reshape(1, _RES)
    yr = y_basis.reshape(1, _RES)
    bT = b.reshape(bsz // 2, 2, _N)
    dT = d.reshape(bsz // 2, 2, _N)
    out = pl.pallas_call(
        _sw_kernel,
        grid=(bsz // 2,),
        in_specs=[
            pl.BlockSpec((1, 2, _N), lambda i: (i, 0, 0)),
            pl.BlockSpec((1, 2, _N), lambda i: (i, 0, 0)),
            pl.BlockSpec((1, _RES), lambda i: (0, 0)),
            pl.BlockSpec((1, _RES), lambda i: (0, 0)),
        ],
        out_specs=pl.BlockSpec((2, _N, _RES), lambda i: (i, 0, 0)),
        out_shape=jax.ShapeDtypeStruct((bsz, _N, _RES), jnp.float32),
        scratch_shapes=[pltpu.VMEM((_N, _W), jnp.float32)],
        compiler_params=pltpu.CompilerParams(
            dimension_semantics=("parallel",),
        ),
    )(bT, dT, xr, yr)
    return out


# speedup vs baseline: 1.1922x; 1.0231x over previous
import jax
import jax.numpy as jnp
from jax.experimental import pallas as pl
from jax.experimental.pallas import tpu as pltpu

_N = 8192
_RES = 64
_C = 128
_P = 256
_NP = _N // _P
_W = 128


def _pass_big(x, j, k, desc):
    R, W = x.shape
    G = R // (2 * j)
    x3 = x.reshape(G, 2 * j, W)
    a = x3[:, :j, :]
    b = x3[:, j:, :]
    mn = jnp.minimum(a, b)
    mx = jnp.maximum(a, b)
    if desc:
        mn, mx = mx, mn
    dirs = [((g * 2 * j) & k) == 0 for g in range(G)]
    if all(dirs):
        first, second = mn, mx
    elif not any(dirs):
        first, second = mx, mn
    else:
        gi = jax.lax.broadcasted_iota(jnp.int32, (G, 1, 1), 0)
        dm = (gi & (k // (2 * j))) == 0
        first = jnp.where(dm, mn, mx)
        second = jnp.where(dm, mx, mn)
    out = jnp.concatenate([first[:, None], second[:, None]], axis=1)
    return out.reshape(R, W)


def _small_uniform(x3, j, asc):
    s = jax.lax.broadcasted_iota(jnp.int32, (1, 8, 1), 1)
    up = (s & j) == 0
    if j == 4:
        p = pltpu.roll(x3, 4, axis=1)
        mn = jnp.minimum(x3, p)
        mx = jnp.maximum(x3, p)
        take = up if asc else jnp.logical_not(up)
        return jnp.where(take, mn, mx)
    r = pltpu.roll(x3, 8 - j, axis=1)
    mn = jnp.minimum(x3, r)
    mx = jnp.maximum(x3, r)
    if not asc:
        mn, mx = mx, mn
    return jnp.where(up, mn, pltpu.roll(mx, j, axis=1))


def _pass_small(x, j, k, desc):
    R, W = x.shape
    G = R // 8
    x3 = x.reshape(G, 8, W)
    if k >= 8:
        L = min(k // 8, G)
        pieces = []
        for r0 in range(0, G, L):
            asc = (((r0 * 8) & k) == 0) != desc
            pieces.append(_small_uniform(x3[r0:r0 + L], j, asc))
        out = pieces[0] if len(pieces) == 1 else jnp.concatenate(pieces, 0)
        return out.reshape(R, W)
    s = jax.lax.broadcasted_iota(jnp.int32, (1, 8, 1), 1)
    up = (s & j) == 0
    p = jnp.where(up, pltpu.roll(x3, 8 - j, axis=1), pltpu.roll(x3, j, axis=1))
    mn = jnp.minimum(x3, p)
    mx = jnp.maximum(x3, p)
    take = up == ((s & k) == 0)
    if desc:
        take = jnp.logical_not(take)
    return jnp.where(take, mn, mx).reshape(R, W)


def _net_pass(x, j, k, desc):
    if j >= 8:
        return _pass_big(x, j, k, desc)
    return _pass_small(x, j, k, desc)


def _local_sort(x, desc):
    R = x.shape[0]
    k = 2
    while k <= R:
        j = k // 2
        while j >= 1:
            x = _net_pass(x, j, k, desc)
            j //= 2
        k *= 2
    return x


def _merge_tail(x, desc):
    R = x.shape[0]
    j = R // 2
    while j >= 1:
        x = _net_pass(x, j, 2 * R, desc)
        j //= 2
    return x


def _sw_kernel(bT_ref, dT_ref, x_ref, y_ref, out_ref, buf):
    xb = x_ref[...]
    yb = y_ref[...]
    z = jnp.zeros_like(xb)
    w4 = jnp.concatenate(
        [
            jnp.concatenate([xb, z], axis=1),
            jnp.concatenate([z, xb], axis=1),
            jnp.concatenate([yb, z], axis=1),
            jnp.concatenate([z, yb], axis=1),
        ],
        axis=0,
    )
    bd = jnp.concatenate([bT_ref[0], dT_ref[0]], axis=0)
    buf[...] = jax.lax.dot_general(
        bd,
        w4,
        (((0,), (0,)), ((), ())),
        precision=jax.lax.Precision.HIGHEST,
        preferred_element_type=jnp.float32,
    )

    def _p1_body(p, desc):
        x = buf[pl.ds(p * _P, _P), :]
        lo = _local_sort(x[:_C, :], desc=False)
        hi = _local_sort(x[_C:, :], desc=True)
        mn = jnp.minimum(lo, hi)
        mx = jnp.maximum(lo, hi)
        if desc:
            mn, mx = mx, mn
        lo = _merge_tail(mn, desc)
        hi = _merge_tail(mx, desc)
        buf[pl.ds(p * _P, _P), :] = jnp.concatenate([lo, hi], axis=0)

    def p1_asc(u, carry):
        _p1_body(4 * u, False)
        _p1_body(4 * u + 2, False)
        return carry

    def p1_desc(u, carry):
        _p1_body(4 * u + 1, True)
        _p1_body(4 * u + 3, True)
        return carry

    jax.lax.fori_loop(0, _NP // 4, p1_asc, 0)
    jax.lax.fori_loop(0, _NP // 4, p1_desc, 0)

    k = 512
    while k <= _N:
        j = k // 2
        while j >= 4 * _P:
            j2 = j // 2
            m = j2 // _P

            def _p2f_one(u, j, j2, m, k):
                g = u // m
                t = u - g * m
                base = g * (2 * j) + t * _P
                A = buf[pl.ds(base, _P), :]
                B = buf[pl.ds(base + j2, _P), :]
                C = buf[pl.ds(base + j, _P), :]
                D = buf[pl.ds(base + j + j2, _P), :]
                mnAC = jnp.minimum(A, C)
                mxAC = jnp.maximum(A, C)
                mnBD = jnp.minimum(B, D)
                mxBD = jnp.maximum(B, D)
                a2 = jnp.minimum(mnAC, mnBD)
                b2 = jnp.maximum(mnAC, mnBD)
                c2 = jnp.minimum(mxAC, mxBD)
                d2 = jnp.maximum(mxAC, mxBD)
                asc = (base & k) == 0

                @pl.when(asc)
                def _():
                    buf[pl.ds(base, _P), :] = a2
                    buf[pl.ds(base + j2, _P), :] = b2
                    buf[pl.ds(base + j, _P), :] = c2
                    buf[pl.ds(base + j + j2, _P), :] = d2

                @pl.when(jnp.logical_not(asc))
                def _():
                    buf[pl.ds(base, _P), :] = d2
                    buf[pl.ds(base + j2, _P), :] = c2
                    buf[pl.ds(base + j, _P), :] = b2
                    buf[pl.ds(base + j + j2, _P), :] = a2

            def p2f_body(u, carry, j=j, j2=j2, m=m, k=k):
                _p2f_one(2 * u, j, j2, m, k)
                _p2f_one(2 * u + 1, j, j2, m, k)
                return carry

            jax.lax.fori_loop(0, _N // (8 * _P), p2f_body, 0)
            j //= 4
        while j >= 2 * _P:
            jb = j // _P

            def _cross_one(u, j=j, jb=jb, k=k):
                g = u // jb
                t = u - g * jb
                base = g * (2 * j) + t * _P
                a = buf[pl.ds(base, _P), :]
                bq = buf[pl.ds(base + j, _P), :]
                mn = jnp.minimum(a, bq)
                mx = jnp.maximum(a, bq)
                asc = (base & k) == 0

                @pl.when(asc)
                def _():
                    buf[pl.ds(base, _P), :] = mn
                    buf[pl.ds(base + j, _P), :] = mx

                @pl.when(jnp.logical_not(asc))
                def _():
                    buf[pl.ds(base, _P), :] = mx
                    buf[pl.ds(base + j, _P), :] = mn

            def p2a_body(u, carry, cross=_cross_one):
                cross(2 * u)
                cross(2 * u + 1)
                return carry

            jax.lax.fori_loop(0, _N // (4 * _P), p2a_body, 0)
            j //= 2

        _Q = 2 * _P
        S = k // _Q
        last = k == _N

        def _tail_body(q, desc, last=last):
            base = q * _Q
            x = buf[pl.ds(base, _Q), :]
            a = x[:_P, :]
            b = x[_P:, :]
            mn = jnp.minimum(a, b)
            mx = jnp.maximum(a, b)
            if desc:
                mn, mx = mx, mn
            halves = []
            for h in (mn, mx):
                lo = h[:_C, :]
                hi = h[_C:, :]
                mn2 = jnp.minimum(lo, hi)
                mx2 = jnp.maximum(lo, hi)
                if desc:
                    mn2, mx2 = mx2, mn2
                halves.append(_merge_tail(mn2, desc))
                halves.append(_merge_tail(mx2, desc))
            res = jnp.concatenate(halves, axis=0)
            if last:
                out_ref[0, pl.ds(base, _Q), :] = res[:, :_RES]
                out_ref[1, pl.ds(base, _Q), :] = res[:, _RES:]
            else:
                buf[pl.ds(base, _Q), :] = res

        def _qmap(u, S=S):
            return (u // S) * 2 * S + (u - (u // S) * S)

        def p2b_asc(u, carry):
            _tail_body(_qmap(2 * u), False)
            _tail_body(_qmap(2 * u + 1), False)
            return carry

        def p2b_desc(u, carry):
            _tail_body(_qmap(2 * u) + S, True)
            _tail_body(_qmap(2 * u + 1) + S, True)
            return carry

        _NQ = _N // _Q
        if last:
            jax.lax.fori_loop(0, _NQ // 2, p2b_asc, 0)
        else:
            jax.lax.fori_loop(0, _NQ // 4, p2b_asc, 0)
            jax.lax.fori_loop(0, _NQ // 4, p2b_desc, 0)
        k *= 2


def kernel(b, d, x_basis, y_basis):
    bsz = b.shape[0]
    xr = x_basis.reshape(1, _RES)
    yr = y_basis.reshape(1, _RES)
    bT = b.reshape(bsz // 2, 2, _N)
    dT = d.reshape(bsz // 2, 2, _N)
    out = pl.pallas_call(
        _sw_kernel,
        grid=(bsz // 2,),
        in_specs=[
            pl.BlockSpec((1, 2, _N), lambda i: (i, 0, 0)),
            pl.BlockSpec((1, 2, _N), lambda i: (i, 0, 0)),
            pl.BlockSpec((1, _RES), lambda i: (0, 0)),
            pl.BlockSpec((1, _RES), lambda i: (0, 0)),
        ],
        out_specs=pl.BlockSpec((2, _N, _RES), lambda i: (i, 0, 0)),
        out_shape=jax.ShapeDtypeStruct((bsz, _N, _RES), jnp.float32),
        scratch_shapes=[pltpu.VMEM((_N, _W), jnp.float32)],
        compiler_params=pltpu.CompilerParams(
            dimension_semantics=("parallel",),
        ),
    )(bT, dT, xr, yr)
    return out
